# Initial kernel scaffold; baseline (speedup 1.0000x reference)
import functools

import jax
import jax.numpy as jnp
from jax import lax
from jax.experimental import pallas as pl
from jax.experimental.pallas import tpu as pltpu
from jax.experimental.pallas import tpu_sc as plsc

N, E, D_IN, D_OUT = 10000, 320000, 128, 128

NC, NS, L = 2, 16, 16
NW = NC * NS
EPW = E // NW
B = 80
NB = EPW // B
ROWS_PER_TILE = N // NS

BLK = 2000


def _stage_a_body(x_ref, w_ref, b_ref, u_ref, h_ref, q_ref, t_ref, q_acc, t_acc):
    i = pl.program_id(0)
    h = lax.dot_general(x_ref[...], w_ref[...], (((1,), (1,)), ((), ())),
                        preferred_element_type=jnp.float32)
    h = h + b_ref[...]
    h_ref[...] = h
    ah = (u_ref[...] * (1.0 / N)) * h
    q_part = jnp.sum(ah * ah)
    t_part = jnp.sum(ah, axis=0, keepdims=True)

    @pl.when(i == 0)
    def _():
        q_acc[0] = q_part
        t_acc[...] = t_part

    @pl.when(i > 0)
    def _():
        q_acc[0] = q_acc[0] + q_part
        t_acc[...] = t_acc[...] + t_part

    @pl.when(i == pl.num_programs(0) - 1)
    def _():
        q_ref[0, 0] = q_acc[0]
        t_ref[...] = t_acc[...]


def _stage_a(x, W, b2, u2):
    return pl.pallas_call(
        _stage_a_body,
        grid=(N // BLK,),
        in_specs=[
            pl.BlockSpec((BLK, D_IN), lambda i: (i, 0)),
            pl.BlockSpec((D_OUT, D_IN), lambda i: (0, 0)),
            pl.BlockSpec((1, D_OUT), lambda i: (0, 0)),
            pl.BlockSpec((BLK, 1), lambda i: (i, 0)),
        ],
        out_specs=[
            pl.BlockSpec((BLK, D_OUT), lambda i: (i, 0)),
            pl.BlockSpec((1, 1), lambda i: (0, 0)),
            pl.BlockSpec((1, D_OUT), lambda i: (0, 0)),
        ],
        out_shape=[
            jax.ShapeDtypeStruct((N, D_OUT), jnp.float32),
            jax.ShapeDtypeStruct((1, 1), jnp.float32),
            jax.ShapeDtypeStruct((1, D_OUT), jnp.float32),
        ],
        scratch_shapes=[
            pltpu.SMEM((1,), jnp.float32),
            pltpu.VMEM((1, D_OUT), jnp.float32),
        ],
    )(x, W, b2, u2)


def _stage_b_body(src_hbm, dst_hbm, w_hbm, h_hbm, out_hbm,
                  src_v, dst_v, w_v, rows_v, y_sh, sem):
    c = lax.axis_index("c")
    s = lax.axis_index("s")
    wid = c * NS + s

    row0 = wid * NB
    pltpu.sync_copy(src_hbm.at[pl.ds(row0, NB)], src_v)
    pltpu.sync_copy(dst_hbm.at[pl.ds(row0, NB)], dst_v)
    pltpu.sync_copy(w_hbm.at[pl.ds(wid * EPW, EPW)], w_v)

    def zero_row(r, carry):
        z = jnp.zeros((L,), jnp.float32)
        for cc in range(D_OUT // L):
            rows_v[r, pl.ds(cc * L, L)] = z
        return carry

    lax.fori_loop(0, B, zero_row, 0)
    base = s * ROWS_PER_TILE
    for k in range(ROWS_PER_TILE // B):
        pltpu.sync_copy(rows_v, y_sh.at[pl.ds(base + k * B, B)])
    tail = ROWS_PER_TILE % B
    if tail:
        pltpu.sync_copy(rows_v.at[pl.ds(0, tail)],
                        y_sh.at[pl.ds(base + (ROWS_PER_TILE // B) * B, tail)])
    plsc.subcore_barrier()

    def batch_body(j, carry):
        pltpu.async_copy(h_hbm.at[src_v.at[j]], rows_v, sem).wait()

        def edge_body(i, icarry):
            wb = plsc.load_gather(w_v, [jnp.full((L,), j * B + i, jnp.int32)])
            for cc in range(D_OUT // L):
                sl = pl.ds(cc * L, L)
                rows_v[i, sl] = rows_v[i, sl] * wb
            return icarry

        lax.fori_loop(0, B, edge_body, 0)
        pltpu.sync_copy(rows_v, y_sh.at[dst_v.at[j]], add=True)
        return carry

    lax.fori_loop(0, NB, batch_body, 0)
    plsc.subcore_barrier()

    pltpu.sync_copy(y_sh.at[pl.ds(base, ROWS_PER_TILE)],
                    out_hbm.at[c, pl.ds(base, ROWS_PER_TILE)])


def _stage_b(src, dst, w_flat, h):
    mesh = plsc.VectorSubcoreMesh(core_axis_name="c", subcore_axis_name="s")
    return pl.kernel(
        _stage_b_body,
        out_type=jax.ShapeDtypeStruct((NC, N, D_OUT), jnp.float32),
        mesh=mesh,
        scratch_types=[
            pltpu.VMEM((NB, B), jnp.int32),
            pltpu.VMEM((NB, B), jnp.int32),
            pltpu.VMEM((EPW,), jnp.float32),
            pltpu.VMEM((B, D_OUT), jnp.float32),
            pltpu.VMEM_SHARED((N, D_OUT), jnp.float32),
            pltpu.SemaphoreType.DMA,
        ],
    )(src, dst, w_flat, h)


def _stage_c_body(yp_ref, q_ref, t_ref, y_ref, reg_ref, s_acc):
    i = pl.program_id(0)
    y = yp_ref[0] + yp_ref[1]
    y_ref[...] = y
    s_part = jnp.sum(y, axis=0, keepdims=True)

    @pl.when(i == 0)
    def _():
        s_acc[...] = s_part

    @pl.when(i > 0)
    def _():
        s_acc[...] = s_acc[...] + s_part

    @pl.when(i == pl.num_programs(0) - 1)
    def _():
        m = s_acc[...] * (1.0 / N)
        cross = jnp.sum(m * t_ref[...])
        msq = jnp.sum(m * m)
        reg_ref[0, 0] = (q_ref[0, 0] - 2.0 * cross + N * msq) * (1.0 / (N * D_OUT))


def _stage_c(ypart, q, t):
    return pl.pallas_call(
        _stage_c_body,
        grid=(N // BLK,),
        in_specs=[
            pl.BlockSpec((NC, BLK, D_OUT), lambda i: (0, i, 0)),
            pl.BlockSpec((1, 1), lambda i: (0, 0)),
            pl.BlockSpec((1, D_OUT), lambda i: (0, 0)),
        ],
        out_specs=[
            pl.BlockSpec((BLK, D_OUT), lambda i: (i, 0)),
            pl.BlockSpec((1, 1), lambda i: (0, 0)),
        ],
        out_shape=[
            jax.ShapeDtypeStruct((N, D_OUT), jnp.float32),
            jax.ShapeDtypeStruct((1, 1), jnp.float32),
        ],
        scratch_shapes=[
            pltpu.VMEM((1, D_OUT), jnp.float32),
        ],
    )(ypart, q, t)


def kernel(x, edge_index, w, u_sum, W, b):
    h, q, t = _stage_a(x, W, b.reshape(1, D_OUT), u_sum.reshape(N, 1))
    src = edge_index[0].reshape(NW * NB, B)
    dst = edge_index[1].reshape(NW * NB, B)
    ypart = _stage_b(src, dst, w.reshape(E), h)
    y, reg = _stage_c(ypart, q, t)
    return y, reg[0, 0]


# speedup vs baseline: 3.6754x; 3.6754x over previous
import jax
import jax.numpy as jnp
from jax import lax
from jax.experimental import pallas as pl
from jax.experimental.pallas import tpu as pltpu
from jax.experimental.pallas import tpu_sc as plsc

N, E, D_IN, D_OUT = 10000, 320000, 128, 128

NC, NS, L = 2, 16, 16
NW = NC * NS
B = 128
NB = 79
EPW = NB * B
EPAD = NW * EPW
RPT = 624
RTAIL = N - RPT * NS

BLK = 2000


def _stage_a_body(x_ref, w_ref, b_ref, u_ref, h_ref, q_ref, t_ref, q_acc, t_acc):
    i = pl.program_id(0)
    h = lax.dot_general(x_ref[...], w_ref[...], (((1,), (1,)), ((), ())),
                        preferred_element_type=jnp.float32)
    h = h + b_ref[...]
    h_ref[...] = h
    ah = (u_ref[...] * (1.0 / N)) * h
    q_part = jnp.sum(ah * ah)
    t_part = jnp.sum(ah, axis=0, keepdims=True)

    @pl.when(i == 0)
    def _():
        q_acc[0] = q_part
        t_acc[...] = t_part

    @pl.when(i > 0)
    def _():
        q_acc[0] = q_acc[0] + q_part
        t_acc[...] = t_acc[...] + t_part

    @pl.when(i == pl.num_programs(0) - 1)
    def _():
        q_ref[...] = jnp.reshape(q_acc[0], (1, 1))
        t_ref[...] = t_acc[...]


def _stage_a(x, W, b2, u2):
    return pl.pallas_call(
        _stage_a_body,
        grid=(N // BLK,),
        in_specs=[
            pl.BlockSpec((BLK, D_IN), lambda i: (i, 0)),
            pl.BlockSpec((D_OUT, D_IN), lambda i: (0, 0)),
            pl.BlockSpec((1, D_OUT), lambda i: (0, 0)),
            pl.BlockSpec((BLK, 1), lambda i: (i, 0)),
        ],
        out_specs=[
            pl.BlockSpec((BLK, D_OUT), lambda i: (i, 0)),
            pl.BlockSpec((1, 1), lambda i: (0, 0)),
            pl.BlockSpec((1, D_OUT), lambda i: (0, 0)),
        ],
        out_shape=[
            jax.ShapeDtypeStruct((N, D_OUT), jnp.float32),
            jax.ShapeDtypeStruct((1, 1), jnp.float32),
            jax.ShapeDtypeStruct((1, D_OUT), jnp.float32),
        ],
        scratch_shapes=[
            pltpu.SMEM((1,), jnp.float32),
            pltpu.VMEM((1, D_OUT), jnp.float32),
        ],
    )(x, W, b2, u2)


def _stage_b_body(src_hbm, dst_hbm, w_hbm, h_hbm, out_hbm,
                  src_v, dst_v, w_v, rows_v, y_sh):
    c = lax.axis_index("c")
    s = lax.axis_index("s")
    wid = c * NS + s

    pltpu.sync_copy(src_hbm.at[wid], src_v)
    pltpu.sync_copy(dst_hbm.at[wid], dst_v)
    pltpu.sync_copy(w_hbm.at[wid], w_v)

    def zero_row(r, carry):
        z = jnp.zeros((L,), jnp.float32)
        for cc in range(D_OUT // L):
            rows_v[r, pl.ds(cc * L, L)] = z
        return carry

    lax.fori_loop(0, B, zero_row, 0)
    base = s * RPT
    for k in range(RPT // B):
        pltpu.sync_copy(rows_v, y_sh.at[pl.ds(base + k * B, B)])
    pltpu.sync_copy(rows_v.at[pl.ds(0, RPT % B)],
                    y_sh.at[pl.ds(base + (RPT // B) * B, RPT % B)])

    @pl.when(s == NS - 1)
    def _():
        pltpu.sync_copy(rows_v.at[pl.ds(0, RTAIL)],
                        y_sh.at[pl.ds(NS * RPT, RTAIL)])

    plsc.subcore_barrier()

    def batch_body(j, carry):
        pltpu.sync_copy(h_hbm.at[src_v.at[j]], rows_v)

        def edge_body(i, icarry):
            wb = plsc.load_gather(w_v, [jnp.full((L,), j * B + i, jnp.int32)])
            for cc in range(D_OUT // L):
                sl = pl.ds(cc * L, L)
                rows_v[i, sl] = rows_v[i, sl] * wb
            return icarry

        lax.fori_loop(0, B, edge_body, 0)
        pltpu.sync_copy(rows_v, y_sh.at[dst_v.at[j]], add=True)
        return carry

    lax.fori_loop(0, NB, batch_body, 0)
    plsc.subcore_barrier()

    pltpu.sync_copy(y_sh.at[pl.ds(base, RPT)],
                    out_hbm.at[c, pl.ds(base, RPT)])

    @pl.when(s == NS - 1)
    def _():
        pltpu.sync_copy(y_sh.at[pl.ds(NS * RPT, RTAIL)],
                        out_hbm.at[c, pl.ds(NS * RPT, RTAIL)])


def _stage_b(src, dst, w_flat, h):
    mesh = plsc.VectorSubcoreMesh(core_axis_name="c", subcore_axis_name="s")
    return pl.kernel(
        _stage_b_body,
        out_type=jax.ShapeDtypeStruct((NC, N, D_OUT), jnp.float32),
        mesh=mesh,
        compiler_params=pltpu.CompilerParams(needs_layout_passes=False),
        scratch_types=[
            pltpu.VMEM((NB, B), jnp.int32),
            pltpu.VMEM((NB, B), jnp.int32),
            pltpu.VMEM((EPW,), jnp.float32),
            pltpu.VMEM((B, D_OUT), jnp.float32),
            pltpu.VMEM_SHARED((N, D_OUT), jnp.float32),
        ],
    )(src, dst, w_flat, h)


def _stage_c_body(yp_ref, q_ref, t_ref, y_ref, reg_ref, s_acc):
    i = pl.program_id(0)
    y = yp_ref[0] + yp_ref[1]
    y_ref[...] = y
    s_part = jnp.sum(y, axis=0, keepdims=True)

    @pl.when(i == 0)
    def _():
        s_acc[...] = s_part

    @pl.when(i > 0)
    def _():
        s_acc[...] = s_acc[...] + s_part

    @pl.when(i == pl.num_programs(0) - 1)
    def _():
        m = s_acc[...] * (1.0 / N)
        cross = jnp.sum(m * t_ref[...])
        msq = jnp.sum(m * m)
        reg_ref[...] = (q_ref[...] - 2.0 * cross + N * msq) * (1.0 / (N * D_OUT))


def _stage_c(ypart, q, t):
    return pl.pallas_call(
        _stage_c_body,
        grid=(N // BLK,),
        in_specs=[
            pl.BlockSpec((NC, BLK, D_OUT), lambda i: (0, i, 0)),
            pl.BlockSpec((1, 1), lambda i: (0, 0)),
            pl.BlockSpec((1, D_OUT), lambda i: (0, 0)),
        ],
        out_specs=[
            pl.BlockSpec((BLK, D_OUT), lambda i: (i, 0)),
            pl.BlockSpec((1, 1), lambda i: (0, 0)),
        ],
        out_shape=[
            jax.ShapeDtypeStruct((N, D_OUT), jnp.float32),
            jax.ShapeDtypeStruct((1, 1), jnp.float32),
        ],
        scratch_shapes=[
            pltpu.VMEM((1, D_OUT), jnp.float32),
        ],
    )(ypart, q, t)


def kernel(x, edge_index, w, u_sum, W, b):
    h, q, t = _stage_a(x, W, b.reshape(1, D_OUT), u_sum.reshape(N, 1))
    pad = EPAD - E
    zi = jnp.zeros((pad,), jnp.int32)
    src = jnp.concatenate([edge_index[0], zi]).reshape(NW, NB, B)
    dst = jnp.concatenate([edge_index[1], zi]).reshape(NW, NB, B)
    wf = jnp.concatenate([w.reshape(E), jnp.zeros((pad,), jnp.float32)])
    ypart = _stage_b(src, dst, wf.reshape(NW, EPW), h)
    y, reg = _stage_c(ypart, q, t)
    return y, reg[0, 0]
